# Initial kernel scaffold; baseline (speedup 1.0000x reference)
import jax
import jax.numpy as jnp
from jax.experimental import pallas as pl


def kernel(x, edge_index, params):
    raise NotImplementedError("write your pallas kernel here")


# speedup vs baseline: 4.6947x; 4.6947x over previous
import functools

import jax
import jax.numpy as jnp
from jax import lax
from jax.experimental import pallas as pl
from jax.experimental.pallas import tpu as pltpu
from jax.experimental.pallas import tpu_sc as plsc

N = 10000
D = 128
H = 64
E = 320000
NUM_CLASSES = 47
BN_EPS = 1e-5

NC = 2
NS = 16
NW = NC * NS
EW = E // NW
K = 80
CH = EW // K
NP = 10240
RT = NP // NS
ZR = 160


def _sc_segsum_body(h0_hbm, h1_hbm, src_hbm, dst_hbm, out_hbm,
                    src_v, dst_v, rows_v, zero_v, agg_sh, sem):
    c = lax.axis_index("c")
    s = lax.axis_index("s")

    pltpu.sync_copy(src_hbm.at[c, s], src_v)
    pltpu.sync_copy(dst_hbm.at[c, s], dst_v)

    zvec = jnp.zeros((16,), jnp.float32)

    def zbody(i, carry):
        for u in range(H // 16):
            zero_v[i, pl.ds(u * 16, 16)] = zvec
        return carry

    lax.fori_loop(0, ZR, zbody, 0)

    for f, h_hbm in enumerate((h0_hbm, h1_hbm)):
        for b in range(RT // ZR):
            pltpu.sync_copy(zero_v, agg_sh.at[pl.ds(s * RT + b * ZR, ZR)])
        plsc.subcore_barrier()

        def ebody(j, carry):
            pltpu.async_copy(h_hbm.at[src_v.at[j]], rows_v, sem).wait()
            pltpu.sync_copy(rows_v, agg_sh.at[dst_v.at[j]], add=True)
            return carry

        lax.fori_loop(0, CH, ebody, 0)
        plsc.subcore_barrier()

        pltpu.sync_copy(agg_sh.at[pl.ds(s * RT, RT)],
                        out_hbm.at[c, f, pl.ds(s * RT, RT)])


@functools.cache
def _sc_segsum_fn():
    return pl.kernel(
        _sc_segsum_body,
        out_type=jax.ShapeDtypeStruct((NC, 2, NP, H), jnp.float32),
        mesh=plsc.VectorSubcoreMesh(core_axis_name="c", subcore_axis_name="s"),
        compiler_params=pltpu.CompilerParams(use_tc_tiling_on_sc=False),
        scratch_types=[
            pltpu.VMEM((CH, K), jnp.int32),
            pltpu.VMEM((CH, K), jnp.int32),
            pltpu.VMEM((K, H), jnp.float32),
            pltpu.VMEM((ZR, H), jnp.float32),
            pltpu.VMEM_SHARED((NP, H), jnp.float32),
            pltpu.SemaphoreType.DMA,
        ],
    )


def _sc_segsum(h0, h1, src4, dst4):
    return _sc_segsum_fn()(h0, h1, src4, dst4)


BR = 1000


def _tc_layer(h0, h1, agg, eps11, w1, b1, w2, b2, scale, beta, *, first, last):
    def body(h0_ref, h1_ref, a00_ref, a01_ref, a10_ref, a11_ref, eps_ref,
             w1_ref, b1_ref, w2_ref, b2_ref, s_ref, bt_ref, *o_refs):
        e1 = 1.0 + eps_ref[0, 0]
        z_lo = h0_ref[...] * e1 + a00_ref[...] + a10_ref[...]
        z_hi = h1_ref[...] * e1 + a01_ref[...] + a11_ref[...]
        z = jnp.concatenate([z_lo, z_hi], axis=1)
        z1 = jnp.dot(z, w1_ref[...], preferred_element_type=jnp.float32)
        z1 = jnp.maximum(z1 + b1_ref[...], 0.0)
        z2 = jnp.dot(z1, w2_ref[...], preferred_element_type=jnp.float32)
        z2 = z2 + b2_ref[...]
        if last:
            o_refs[0][...] = z2
        else:
            z2 = jnp.maximum(z2 * s_ref[...] + bt_ref[...], 0.0)
            if not first:
                z2 = z2 + jnp.concatenate([h0_ref[...], h1_ref[...]], axis=1)
            o_refs[0][...] = z2[:, :H]
            o_refs[1][...] = z2[:, H:]

    half_spec = pl.BlockSpec((BR, H), lambda i: (i, 0))
    full = lambda shp: pl.BlockSpec(shp, lambda i: (0,) * len(shp))
    if last:
        out_shape = jax.ShapeDtypeStruct((N, D), jnp.float32)
        out_specs = pl.BlockSpec((BR, D), lambda i: (i, 0))
    else:
        out_shape = (jax.ShapeDtypeStruct((N, H), jnp.float32),
                     jax.ShapeDtypeStruct((N, H), jnp.float32))
        out_specs = (half_spec, half_spec)
    return pl.pallas_call(
        body,
        grid=(N // BR,),
        in_specs=[
            half_spec, half_spec,
            half_spec, half_spec, half_spec, half_spec,
            full((1, 1)),
            full((D, D)), full((1, D)),
            full((D, D)), full((1, D)),
            full((1, D)), full((1, D)),
        ],
        out_specs=out_specs,
        out_shape=out_shape,
    )(h0, h1, agg[0, 0], agg[0, 1], agg[1, 0], agg[1, 1],
      eps11, w1, b1, w2, b2, scale, beta)


def kernel(x, edge_index, params):
    src4 = edge_index[0].reshape(NC, NS, CH, K)
    dst4 = edge_index[1].reshape(NC, NS, CH, K)
    n_layers = len(params)
    h0 = x[:, :H]
    h1 = x[:, H:]
    for i, p in enumerate(params):
        agg = _sc_segsum(h0, h1, src4, dst4)
        last = i == n_layers - 1
        w2 = p["W2"]
        b2 = p["b2"]
        if last:
            w2 = jnp.pad(w2, ((0, 0), (0, D - NUM_CLASSES)))
            b2 = jnp.pad(b2, (0, D - NUM_CLASSES))
            scale = jnp.ones((D,), jnp.float32)
            beta = jnp.zeros((D,), jnp.float32)
        else:
            scale = p["gamma"] * (1.0 / (1.0 + BN_EPS) ** 0.5)
            beta = p["beta"]
        out = _tc_layer(
            h0, h1, agg, p["eps"].reshape(1, 1),
            p["W1"], p["b1"].reshape(1, -1), w2, b2.reshape(1, -1),
            scale.reshape(1, -1), beta.reshape(1, -1),
            first=(i == 0), last=last)
        if not last:
            h0, h1 = out
    return out[:, :NUM_CLASSES]


# speedup vs baseline: 7.4473x; 1.5863x over previous
import functools

import jax
import jax.numpy as jnp
from jax import lax
from jax.experimental import pallas as pl
from jax.experimental.pallas import tpu as pltpu
from jax.experimental.pallas import tpu_sc as plsc

N = 10000
D = 128
H = 64
E = 320000
NUM_CLASSES = 47
BN_EPS = 1e-5

NC = 2
NS = 16
NW = NC * NS
EW = E // NW
K = 80
CH = EW // K
NP = 10240
RT = NP // NS
ZR = 160


def _sc_segsum_body(h0_hbm, h1_hbm, src_hbm, dst_hbm, out_hbm,
                    src_v, dst_v, rows0_v, rows1_v, zero_v, agg_sh,
                    sem0, sem1):
    c = lax.axis_index("c")
    s = lax.axis_index("s")

    pltpu.sync_copy(src_hbm.at[c, s], src_v)
    pltpu.sync_copy(dst_hbm.at[c, s], dst_v)

    zvec = jnp.zeros((16,), jnp.float32)

    def zbody(i, carry):
        for u in range(H // 16):
            zero_v[i, pl.ds(u * 16, 16)] = zvec
        return carry

    lax.fori_loop(0, ZR, zbody, 0)

    for f, h_hbm in enumerate((h0_hbm, h1_hbm)):
        for b in range(RT // ZR):
            pltpu.sync_copy(zero_v, agg_sh.at[pl.ds(s * RT + b * ZR, ZR)])
        plsc.subcore_barrier()

        pltpu.async_copy(h_hbm.at[src_v.at[0]], rows0_v, sem0)

        def ebody(j, carry):
            def halfstep(rows_a, sem_a, rows_b, sem_b):
                @pl.when(j + 1 < CH)
                def _():
                    pltpu.async_copy(h_hbm.at[src_v.at[j + 1]], rows_b, sem_b)

                pltpu.make_async_copy(h_hbm.at[src_v.at[j]], rows_a,
                                      sem_a).wait()
                pltpu.sync_copy(rows_a, agg_sh.at[dst_v.at[j]], add=True)

            @pl.when(j % 2 == 0)
            def _():
                halfstep(rows0_v, sem0, rows1_v, sem1)

            @pl.when(j % 2 == 1)
            def _():
                halfstep(rows1_v, sem1, rows0_v, sem0)

            return carry

        lax.fori_loop(0, CH, ebody, 0)
        plsc.subcore_barrier()

        pltpu.sync_copy(agg_sh.at[pl.ds(s * RT, RT)],
                        out_hbm.at[c, f, pl.ds(s * RT, RT)])


@functools.cache
def _sc_segsum_fn():
    return pl.kernel(
        _sc_segsum_body,
        out_type=jax.ShapeDtypeStruct((NC, 2, NP, H), jnp.float32),
        mesh=plsc.VectorSubcoreMesh(core_axis_name="c", subcore_axis_name="s"),
        compiler_params=pltpu.CompilerParams(use_tc_tiling_on_sc=False),
        scratch_types=[
            pltpu.VMEM((CH, K), jnp.int32),
            pltpu.VMEM((CH, K), jnp.int32),
            pltpu.VMEM((K, H), jnp.float32),
            pltpu.VMEM((K, H), jnp.float32),
            pltpu.VMEM((ZR, H), jnp.float32),
            pltpu.VMEM_SHARED((NP, H), jnp.float32),
            pltpu.SemaphoreType.DMA,
            pltpu.SemaphoreType.DMA,
        ],
    )


def _sc_segsum(h0, h1, src4, dst4):
    return _sc_segsum_fn()(h0, h1, src4, dst4)


BR = 1000


def _tc_layer(h0, h1, agg, eps11, w1, b1, w2, b2, scale, beta, *, first, last):
    def body(h0_ref, h1_ref, a00_ref, a01_ref, a10_ref, a11_ref, eps_ref,
             w1_ref, b1_ref, w2_ref, b2_ref, s_ref, bt_ref, *o_refs):
        e1 = 1.0 + eps_ref[0, 0]
        z_lo = h0_ref[...] * e1 + a00_ref[...] + a10_ref[...]
        z_hi = h1_ref[...] * e1 + a01_ref[...] + a11_ref[...]
        z = jnp.concatenate([z_lo, z_hi], axis=1)
        z1 = jnp.dot(z, w1_ref[...], preferred_element_type=jnp.float32)
        z1 = jnp.maximum(z1 + b1_ref[...], 0.0)
        z2 = jnp.dot(z1, w2_ref[...], preferred_element_type=jnp.float32)
        z2 = z2 + b2_ref[...]
        if last:
            o_refs[0][...] = z2
        else:
            z2 = jnp.maximum(z2 * s_ref[...] + bt_ref[...], 0.0)
            if not first:
                z2 = z2 + jnp.concatenate([h0_ref[...], h1_ref[...]], axis=1)
            o_refs[0][...] = z2[:, :H]
            o_refs[1][...] = z2[:, H:]

    half_spec = pl.BlockSpec((BR, H), lambda i: (i, 0))
    full = lambda shp: pl.BlockSpec(shp, lambda i: (0,) * len(shp))
    if last:
        out_shape = jax.ShapeDtypeStruct((N, D), jnp.float32)
        out_specs = pl.BlockSpec((BR, D), lambda i: (i, 0))
    else:
        out_shape = (jax.ShapeDtypeStruct((N, H), jnp.float32),
                     jax.ShapeDtypeStruct((N, H), jnp.float32))
        out_specs = (half_spec, half_spec)
    return pl.pallas_call(
        body,
        grid=(N // BR,),
        in_specs=[
            half_spec, half_spec,
            half_spec, half_spec, half_spec, half_spec,
            full((1, 1)),
            full((D, D)), full((1, D)),
            full((D, D)), full((1, D)),
            full((1, D)), full((1, D)),
        ],
        out_specs=out_specs,
        out_shape=out_shape,
    )(h0, h1, agg[0, 0], agg[0, 1], agg[1, 0], agg[1, 1],
      eps11, w1, b1, w2, b2, scale, beta)


def kernel(x, edge_index, params):
    src4 = edge_index[0].reshape(NC, NS, CH, K)
    dst4 = edge_index[1].reshape(NC, NS, CH, K)
    n_layers = len(params)
    h0 = x[:, :H]
    h1 = x[:, H:]
    for i, p in enumerate(params):
        agg = _sc_segsum(h0, h1, src4, dst4)
        last = i == n_layers - 1
        w2 = p["W2"]
        b2 = p["b2"]
        if last:
            w2 = jnp.pad(w2, ((0, 0), (0, D - NUM_CLASSES)))
            b2 = jnp.pad(b2, (0, D - NUM_CLASSES))
            scale = jnp.ones((D,), jnp.float32)
            beta = jnp.zeros((D,), jnp.float32)
        else:
            scale = p["gamma"] * (1.0 / (1.0 + BN_EPS) ** 0.5)
            beta = p["beta"]
        out = _tc_layer(
            h0, h1, agg, p["eps"].reshape(1, 1),
            p["W1"], p["b1"].reshape(1, -1), w2, b2.reshape(1, -1),
            scale.reshape(1, -1), beta.reshape(1, -1),
            first=(i == 0), last=last)
        if not last:
            h0, h1 = out
    return out[:, :NUM_CLASSES]
